# Initial kernel scaffold; baseline (speedup 1.0000x reference)
import functools

import jax
import jax.numpy as jnp
from jax import lax
from jax.experimental import pallas as pl
from jax.experimental.pallas import tpu as pltpu
from jax.experimental.pallas import tpu_sc as plsc

N_NODES = 10000
D = 128
E_EDGES = 320000

NC = 2
NS = 16
NW = NC * NS
CHUNK = 128
NCH = 79
EPT = NCH * CHUNK
E_PAD = EPT * NW
ACC_N = 10240
CW = 16
ROWS_PT = N_NODES // NS
ZPT = ACC_N // NS

_vector_mesh = plsc.VectorSubcoreMesh(core_axis_name="c", subcore_axis_name="s")


@functools.partial(
    pl.kernel,
    out_type=(
        jax.ShapeDtypeStruct((NC, N_NODES, D), jnp.float32),
        jax.ShapeDtypeStruct((NC, N_NODES, CW), jnp.float32),
    ),
    mesh=_vector_mesh,
    scratch_types=[
        pltpu.VMEM((NCH, CHUNK), jnp.int32),
        pltpu.VMEM((NCH, CHUNK), jnp.int32),
        pltpu.VMEM((CHUNK, D), jnp.float32),
        pltpu.VMEM((CHUNK, CW), jnp.float32),
        pltpu.VMEM_SHARED((ACC_N, D), jnp.float32),
        pltpu.VMEM_SHARED((ACC_N, CW), jnp.float32),
    ],
)
def _sc_aggregate(x_hbm, src_hbm, dst_hbm, sum_hbm, cnt_hbm,
                  src_v, dst_v, rows_v, ones_v, acc_sh, cnt_sh):
    cid = lax.axis_index("c")
    sid = lax.axis_index("s")
    wid = cid * NS + sid

    @pl.loop(0, CHUNK)
    def _(i):
        @pl.loop(0, D, step=16)
        def _(j):
            rows_v[i, pl.ds(j, 16)] = jnp.zeros((16,), jnp.float32)
        ones_v[i, pl.ds(0, 16)] = jnp.zeros((16,), jnp.float32)

    zbase = sid * ZPT
    @pl.loop(0, ZPT // CHUNK)
    def _(k):
        pltpu.sync_copy(rows_v, acc_sh.at[pl.ds(zbase + k * CHUNK, CHUNK)])
        pltpu.sync_copy(ones_v, cnt_sh.at[pl.ds(zbase + k * CHUNK, CHUNK)])

    @pl.loop(0, CHUNK)
    def _(i):
        ones_v[i, pl.ds(0, 16)] = jnp.ones((16,), jnp.float32)

    pltpu.sync_copy(src_hbm.at[wid], src_v)
    pltpu.sync_copy(dst_hbm.at[wid], dst_v)

    plsc.subcore_barrier()

    @pl.loop(0, NCH)
    def _(j):
        pltpu.sync_copy(x_hbm.at[src_v.at[j]], rows_v)
        pltpu.sync_copy(rows_v, acc_sh.at[dst_v.at[j]], add=True)
        pltpu.sync_copy(ones_v, cnt_sh.at[dst_v.at[j]], add=True)

    plsc.subcore_barrier()

    obase = sid * ROWS_PT
    pltpu.sync_copy(acc_sh.at[pl.ds(obase, ROWS_PT)],
                    sum_hbm.at[cid].at[pl.ds(obase, ROWS_PT)])
    pltpu.sync_copy(cnt_sh.at[pl.ds(obase, ROWS_PT)],
                    cnt_hbm.at[cid].at[pl.ds(obase, ROWS_PT)])


BLK = 400


def _combine_body(sum_ref, cnt_ref, x_ref, wl_ref, wr_ref, bl_ref, o_ref):
    s = sum_ref[0] + sum_ref[1]
    c = cnt_ref[0, :, 0] + cnt_ref[1, :, 0]
    mean = s / jnp.maximum(c, 1.0)[:, None]
    o_ref[...] = (
        jnp.dot(mean, wl_ref[...], preferred_element_type=jnp.float32)
        + jnp.dot(x_ref[...], wr_ref[...], preferred_element_type=jnp.float32)
        + bl_ref[...]
    )


def _combine(sums, cnts, x, wl_t, wr_t, bl):
    return pl.pallas_call(
        _combine_body,
        grid=(N_NODES // BLK,),
        in_specs=[
            pl.BlockSpec((NC, BLK, D), lambda i: (0, i, 0)),
            pl.BlockSpec((NC, BLK, CW), lambda i: (0, i, 0)),
            pl.BlockSpec((BLK, D), lambda i: (i, 0)),
            pl.BlockSpec((D, D), lambda i: (0, 0)),
            pl.BlockSpec((D, D), lambda i: (0, 0)),
            pl.BlockSpec((1, D), lambda i: (0, 0)),
        ],
        out_specs=pl.BlockSpec((BLK, D), lambda i: (i, 0)),
        out_shape=jax.ShapeDtypeStruct((N_NODES, D), jnp.float32),
    )(sums, cnts, x, wl_t, wr_t, bl)


def kernel(x, edge_index, W_l, b_l, W_r):
    src = edge_index[0]
    dst = edge_index[1]
    pad = E_PAD - E_EDGES
    src_p = jnp.concatenate([src, jnp.zeros((pad,), jnp.int32)])
    dst_p = jnp.concatenate([dst, jnp.full((pad,), N_NODES, jnp.int32)])
    src_r = src_p.reshape(NW, NCH, CHUNK)
    dst_r = dst_p.reshape(NW, NCH, CHUNK)

    sums, cnts = _sc_aggregate(x, src_r, dst_r)

    return _combine(sums, cnts, x, W_l.T, W_r.T, b_l.reshape(1, D))


# speedup vs baseline: 5.5621x; 5.5621x over previous
import functools

import jax
import jax.numpy as jnp
from jax import lax
from jax.experimental import pallas as pl
from jax.experimental.pallas import tpu as pltpu
from jax.experimental.pallas import tpu_sc as plsc

N_NODES = 10000
D = 128
DA = 136
E_EDGES = 320000

NC = 2
NS = 16
NW = NC * NS
CHUNK = 128
NCH = 79
EPT = NCH * CHUNK
E_PAD = EPT * NW
ACC_N = 10112
ZPT = ACC_N // NS

_vector_mesh = plsc.VectorSubcoreMesh(core_axis_name="c", subcore_axis_name="s")


@functools.partial(
    pl.kernel,
    out_type=jax.ShapeDtypeStruct((NC, ACC_N, DA), jnp.float32),
    mesh=_vector_mesh,
    scratch_types=[
        pltpu.VMEM((NCH, CHUNK), jnp.int32),
        pltpu.VMEM((NCH, CHUNK), jnp.int32),
        pltpu.VMEM((CHUNK, DA), jnp.float32),
        pltpu.VMEM_SHARED((ACC_N, DA), jnp.float32),
    ],
    compiler_params=pltpu.CompilerParams(use_tc_tiling_on_sc=False),
)
def _sc_aggregate(xa_hbm, src_hbm, dst_hbm, zero_hbm, sum_hbm,
                  src_v, dst_v, rows_v, acc_sh):
    cid = lax.axis_index("c")
    sid = lax.axis_index("s")
    wid = cid * NS + sid

    zbase = sid * ZPT
    pltpu.sync_copy(zero_hbm, acc_sh.at[pl.ds(zbase, ZPT)])
    pltpu.sync_copy(src_hbm.at[wid], src_v)
    pltpu.sync_copy(dst_hbm.at[wid], dst_v)

    plsc.subcore_barrier()

    @pl.loop(0, NCH)
    def _(j):
        pltpu.sync_copy(xa_hbm.at[src_v.at[j]], rows_v)
        pltpu.sync_copy(rows_v, acc_sh.at[dst_v.at[j]], add=True)

    plsc.subcore_barrier()

    pltpu.sync_copy(acc_sh.at[pl.ds(zbase, ZPT)],
                    sum_hbm.at[cid].at[pl.ds(zbase, ZPT)])


BLK = 400


def _combine_body(sum_ref, x_ref, wl_ref, wr_ref, bl_ref, o_ref):
    a = sum_ref[0] + sum_ref[1]
    s = a[:, :D]
    c = a[:, D:D + 1]
    mean = s / jnp.maximum(c, 1.0)
    o_ref[...] = (
        jnp.dot(mean, wl_ref[...], preferred_element_type=jnp.float32)
        + jnp.dot(x_ref[...], wr_ref[...], preferred_element_type=jnp.float32)
        + bl_ref[...]
    )


def _combine(sums, x, wl_t, wr_t, bl):
    return pl.pallas_call(
        _combine_body,
        grid=(N_NODES // BLK,),
        in_specs=[
            pl.BlockSpec((NC, BLK, DA), lambda i: (0, i, 0)),
            pl.BlockSpec((BLK, D), lambda i: (i, 0)),
            pl.BlockSpec((D, D), lambda i: (0, 0)),
            pl.BlockSpec((D, D), lambda i: (0, 0)),
            pl.BlockSpec((1, D), lambda i: (0, 0)),
        ],
        out_specs=pl.BlockSpec((BLK, D), lambda i: (i, 0)),
        out_shape=jax.ShapeDtypeStruct((N_NODES, D), jnp.float32),
    )(sums, x, wl_t, wr_t, bl)


def kernel(x, edge_index, W_l, b_l, W_r):
    src = edge_index[0]
    dst = edge_index[1]
    pad = E_PAD - E_EDGES
    src_p = jnp.concatenate([src, jnp.zeros((pad,), jnp.int32)])
    dst_p = jnp.concatenate([dst, jnp.full((pad,), N_NODES, jnp.int32)])
    src_r = src_p.reshape(NW, NCH, CHUNK)
    dst_r = dst_p.reshape(NW, NCH, CHUNK)

    xa = jnp.concatenate(
        [x, jnp.ones((N_NODES, 1), jnp.float32),
         jnp.zeros((N_NODES, DA - D - 1), jnp.float32)], axis=1)
    zeros = jnp.zeros((ZPT, DA), jnp.float32)

    sums = _sc_aggregate(xa, src_r, dst_r, zeros)

    return _combine(sums, x, W_l.T, W_r.T, b_l.reshape(1, D))
